# baseline (speedup 1.0000x reference)
import math
from itertools import permutations as _permutations

import jax
import jax.numpy as jnp
import numpy as np
from jax import lax
from jax.experimental import pallas as pl
from jax.experimental.pallas import tpu as pltpu
from jax.experimental.pallas import tpu_sc as plsc

_L = 8
_S = 4
_D = 3
_ND = 128
_ED = 16
_N = 10000
_P = 24
_PERMS = np.array(list(_permutations(range(_S))), dtype=np.int32)

_M = math.pi / 2

_C = 64
_NCHUNK = -(-_N // _C)
_NW = 32
_PART = 1088

_ATAN_C = (0.9999999227745398, -0.3333223244657235, 0.19974024787565844,
           -0.14047793148813997, 0.10002110154691828, -0.060872867201036907,
           0.02533036269905139, -0.005020633432245819)


def _atan_pos(y):
    big = y > 1.0
    z = jnp.where(big, 1.0 / jnp.maximum(y, 1e-30), y)
    u = z * z
    p = jnp.full_like(u, _ATAN_C[-1])
    for c in _ATAN_C[-2::-1]:
        p = p * u + jnp.float32(c)
    a = z * p
    return jnp.where(big, jnp.float32(_M) - a, a)


def _sc_sqrt(x):
    i = lax.bitcast_convert_type(x, jnp.int32)
    i = jnp.int32(0x5F3759DF) - lax.shift_right_logical(i, 1)
    y = lax.bitcast_convert_type(i, jnp.float32)
    for _ in range(3):
        y = y * (jnp.float32(1.5) - jnp.float32(0.5) * x * y * y)
    return jnp.where(x > 0, x * y, jnp.float32(0.0))


def _sc_body(xn_hbm, xf_hbm, ed_hbm, pn_hbm, pf_hbm, out_hbm,
             xnb, xfb, edb, pnb, pfb, part, sem):
    wid = lax.axis_index("s") * 2 + lax.axis_index("c")
    zero = jnp.zeros((16,), jnp.float32)
    for j in range(_PART // 16):
        part[pl.ds(16 * j, 16)] = zero

    lane = lax.iota(jnp.int32, 16)

    def row_body(r, regs):
        xq, fq, eq = list(regs[0]), list(regs[1]), list(regs[2])
        b = r * 512
        for j in range(32):
            v = xnb[pl.ds(b + 16 * j, 16)]
            plsc.addupdate(part.at[pl.ds(16 * j, 16)], v)
            xq[j % 4] = xq[j % 4] + v * v
        b = r * 128
        for j in range(8):
            v = xfb[pl.ds(b + 16 * j, 16)]
            plsc.addupdate(part.at[pl.ds(512 + 16 * j, 16)], v)
            fq[j % 2] = fq[j % 2] + v * v
        b = r * 64
        for j in range(4):
            v = edb[pl.ds(b + 16 * j, 16)]
            plsc.addupdate(part.at[pl.ds(640 + 16 * j, 16)], v)
            eq[j % 2] = eq[j % 2] + v * v
        return (tuple(xq), tuple(fq), tuple(eq))

    def chunk_body(k, regs):
        ck = wid + _NW * k
        row0 = jnp.minimum(ck * _C, _N - _C)
        local0 = ck * _C - row0
        hs = [
            pltpu.async_copy(xn_hbm.at[pl.ds(row0 * 512, _C * 512)], xnb, sem),
            pltpu.async_copy(xf_hbm.at[pl.ds(row0 * 128, _C * 128)], xfb, sem),
            pltpu.async_copy(ed_hbm.at[pl.ds(row0 * 64, _C * 64)], edb, sem),
        ]
        for r in range(12):
            hs.append(pltpu.async_copy(
                pn_hbm.at[pl.ds(r * _N + row0, _C)],
                pnb.at[pl.ds(r * _C, _C)], sem))
        for r in range(3):
            hs.append(pltpu.async_copy(
                pf_hbm.at[pl.ds(r * _N + row0, _C)],
                pfb.at[pl.ds(r * _C, _C)], sem))
        for h in hs:
            h.wait()

        regs = lax.fori_loop(local0, _C, row_body, regs)

        for g in range(4):
            ok = (lane + 16 * g) >= local0
            pf_v = [pfb[pl.ds(d * _C + 16 * g, 16)] for d in range(3)]
            pe = [pnb[pl.ds((s * 3 + d) * _C + 16 * g, 16)] - pf_v[d]
                  for s in range(4) for d in range(3)]
            ssq = [pe[3 * s] * pe[3 * s] + pe[3 * s + 1] * pe[3 * s + 1]
                   + pe[3 * s + 2] * pe[3 * s + 2] for s in range(4)]
            na = [_sc_sqrt(q) for q in ssq]
            for s in range(4):
                sp = (s - 1) % 4
                dot = (pe[3 * sp] * pe[3 * s] + pe[3 * sp + 1] * pe[3 * s + 1]
                       + pe[3 * sp + 2] * pe[3 * s + 2])
                cosv = dot / jnp.maximum(na[sp] * na[s], jnp.float32(1e-8))
                iv = jnp.where(ok, cosv, jnp.float32(0.0))
                lv = jnp.where(ok, na[s], jnp.float32(0.0))
                plsc.addupdate(part.at[pl.ds(704 + 16 * s, 16)], iv)
                plsc.addupdate(part.at[pl.ds(768 + 16 * s, 16)], lv)
                plsc.addupdate(part.at[pl.ds(832 + 16 * s, 16)], iv * iv)
                plsc.addupdate(part.at[pl.ds(896 + 16 * s, 16)], lv * lv)
        return regs

    regs0 = ((zero,) * 4, (zero,) * 2, (zero,) * 2)
    nk = jnp.where(wid < _NCHUNK - 4 * _NW, 5, 4)
    regs = lax.fori_loop(0, nk, chunk_body, regs0)

    for i in range(4):
        part[pl.ds(960 + 16 * i, 16)] = regs[0][i]
    for i in range(2):
        part[pl.ds(1024 + 16 * i, 16)] = regs[1][i]
    for i in range(2):
        part[pl.ds(1056 + 16 * i, 16)] = regs[2][i]
    pltpu.sync_copy(part, out_hbm.at[pl.ds(wid * _PART, _PART)])


def _sc_stats(xn1, xf1, ed1, pn1, pf1):
    mesh = plsc.VectorSubcoreMesh(core_axis_name="c", subcore_axis_name="s")
    f = pl.kernel(
        _sc_body,
        mesh=mesh,
        out_type=jax.ShapeDtypeStruct((_NW * _PART,), jnp.float32),
        scratch_types=[
            pltpu.VMEM((_C * 512,), jnp.float32),
            pltpu.VMEM((_C * 128,), jnp.float32),
            pltpu.VMEM((_C * 64,), jnp.float32),
            pltpu.VMEM((12 * _C,), jnp.float32),
            pltpu.VMEM((3 * _C,), jnp.float32),
            pltpu.VMEM((_PART,), jnp.float32),
            pltpu.SemaphoreType.DMA,
        ],
    )
    return f(xn1, xf1, ed1, pn1, pf1)


def _intra_cols(p12):
    cur = [p12[:, 3 * s:3 * s + 3] for s in range(_S)]
    ssq = [jnp.sum(c * c, axis=-1, keepdims=True) for c in cur]
    na = [jnp.sqrt(q) for q in ssq]
    intra = []
    for s in range(_S):
        sp = (s - 1) % _S
        dot = jnp.sum(cur[sp] * cur[s], axis=-1, keepdims=True)
        intra.append(dot / jnp.maximum(na[sp] * na[s], 1e-8))
    return jnp.concatenate(intra, axis=-1), jnp.concatenate(na, axis=-1)


def _epi_body(st_ref, pxs_ref, ped_ref, pps_ref, xc_ref, out_ref):
    nf = jnp.float32(_N)
    st = st_ref[...]
    s = jnp.sum(st, axis=0, keepdims=True)
    s_xn = s[:, 0:512]
    s_xf = s[:, 512:640]
    s_ed = s[:, 640:704]
    a_in = [jnp.sum(s[:, 704 + 16 * k:720 + 16 * k]) for k in range(4)]
    a_ln = [jnp.sum(s[:, 768 + 16 * k:784 + 16 * k]) for k in range(4)]
    q_in = jnp.sum(s[:, 832:896])
    q_ln = jnp.sum(s[:, 896:960])
    q_xn = jnp.sum(s[:, 960:1024])
    q_xf = jnp.sum(s[:, 1024:1056])
    q_ed = jnp.sum(s[:, 1056:1088])

    iota = lax.broadcasted_iota(jnp.int32, (_P, 1), 0)
    ot = jnp.zeros((1, _L), jnp.float32)
    oi = lax.broadcasted_iota(jnp.int32, (1, _L), 1)
    for l in range(_L):
        pxs = pxs_ref[pl.ds(_P * l, _P), :]
        ped = ped_ref[pl.ds(_P * l, _P), :]
        pps = pps_ref[pl.ds(_P * l, _P), :]

        b_in, b_ln = _intra_cols(pps)
        cr_in = sum(b_in[:, k:k + 1] * a_in[k] for k in range(4))
        cr_ln = sum(b_ln[:, k:k + 1] * a_ln[k] for k in range(4))
        t_ang = (q_in - 2.0 * cr_in
                 + nf * jnp.sum(b_in * b_in, -1, keepdims=True))
        t_len = (q_ln - 2.0 * cr_ln
                 + nf * jnp.sum(b_ln * b_ln, -1, keepdims=True))
        t_sup = (q_xn - 2.0 * jnp.sum(pxs * s_xn, -1, keepdims=True)
                 + nf * jnp.sum(pxs * pxs, -1, keepdims=True))
        t_edg = (q_ed - 2.0 * jnp.sum(ped * s_ed, -1, keepdims=True)
                 + nf * jnp.sum(ped * ped, -1, keepdims=True))

        tmin = jnp.min(t_ang)
        bidx = jnp.min(jnp.where(t_ang <= tmin, iota, _P))
        onehot = iota == bidx
        t_len_b = jnp.sum(jnp.where(onehot, t_len, 0.0))
        t_sup_b = jnp.sum(jnp.where(onehot, t_sup, 0.0))
        t_edg_b = jnp.sum(jnp.where(onehot, t_edg, 0.0))

        xc = xc_ref[pl.ds(l, 1), :]
        t_cen = q_xf - 2.0 * jnp.sum(xc * s_xf) + nf * jnp.sum(xc * xc)

        sc_ang = _atan_pos(1.0 / tmin)
        sc_len = _atan_pos(1.0 / t_len_b)
        sc_sup = _atan_pos(1.0 / t_sup_b)
        sc_cen = _atan_pos(1.0 / t_cen)
        sc_edg = _atan_pos(1.0 / t_edg_b)

        m = jnp.float32(_M)
        tot = ((sc_len - m) ** 2 + (sc_ang - m) ** 2 + (sc_sup - m) ** 2
               + (sc_cen - m) ** 2 + (sc_edg - m) ** 2)
        sc = _atan_pos(1.0 / tot)
        ot = ot + jnp.where(oi == l, sc, 0.0)
    out_ref[...] = ot


def _epilogue(stats, pxs, ped, pps, xc2, interpret=False):
    return pl.pallas_call(
        _epi_body,
        out_shape=jax.ShapeDtypeStruct((1, _L), jnp.float32),
        interpret=interpret,
    )(stats, pxs, ped, pps, xc2)


def kernel(x_focal, p_focal, x_neighbor, p_neighbor, edge_attr_neighbor,
           x_center, x_support, edge_attr_support, p_support):
    n = x_focal.shape[0]
    xn1 = x_neighbor.reshape(n * _S * _ND)
    xf1 = x_focal.reshape(n * _ND)
    ed1 = edge_attr_neighbor.reshape(n * _S * _ED)
    pn1 = p_neighbor.reshape(n, _S * _D).T.reshape(-1)
    pf1 = p_focal.T.reshape(-1)

    stats = _sc_stats(xn1, xf1, ed1, pn1, pf1).reshape(_NW, _PART)

    pxs = x_support[:, _PERMS].reshape(_L * _P, _S * _ND)
    ped = edge_attr_support[:, _PERMS].reshape(_L * _P, _S * _ED)
    pps = p_support[:, _PERMS].reshape(_L * _P, _S * _D)
    xc2 = x_center.reshape(_L, _ND)
    out = _epilogue(stats, pxs, ped, pps, xc2)
    return out.reshape(_L)


# speedup vs baseline: 2.0463x; 2.0463x over previous
import math
from itertools import permutations as _permutations

import jax
import jax.numpy as jnp
import numpy as np
from jax import lax
from jax.experimental import pallas as pl
from jax.experimental.pallas import tpu as pltpu
from jax.experimental.pallas import tpu_sc as plsc

_L = 8
_S = 4
_D = 3
_ND = 128
_ED = 16
_N = 10000
_P = 24
_PERMS = np.array(list(_permutations(range(_S))), dtype=np.int32)

_M = math.pi / 2

_NPAD = 10240
_WROWS = _NPAD // 32
_GEOM = 256

_BLK = 1000
_G = _N // _BLK

_ATAN_C = (0.9999999227745398, -0.3333223244657235, 0.19974024787565844,
           -0.14047793148813997, 0.10002110154691828, -0.060872867201036907,
           0.02533036269905139, -0.005020633432245819)


def _atan_pos(y):
    big = y > 1.0
    z = jnp.where(big, 1.0 / jnp.maximum(y, 1e-30), y)
    u = z * z
    p = jnp.full_like(u, _ATAN_C[-1])
    for c in _ATAN_C[-2::-1]:
        p = p * u + jnp.float32(c)
    a = z * p
    return jnp.where(big, jnp.float32(_M) - a, a)


def _sc_sqrt(x):
    i = lax.bitcast_convert_type(x, jnp.int32)
    i = jnp.int32(0x5F3759DF) - lax.shift_right_logical(i, 1)
    y = lax.bitcast_convert_type(i, jnp.float32)
    for _ in range(3):
        y = y * (jnp.float32(1.5) - jnp.float32(0.5) * x * y * y)
    return jnp.where(x > 0, x * y, jnp.float32(0.0))


def _sc_geom_body(pc_hbm, out_hbm, pcb, part, sem):
    wid = lax.axis_index("s") * 2 + lax.axis_index("c")
    n0 = wid * _WROWS
    hs = [pltpu.async_copy(pc_hbm.at[pl.ds(r * _NPAD + n0, _WROWS)],
                           pcb.at[pl.ds(r * _WROWS, _WROWS)], sem)
          for r in range(15)]
    zero = jnp.zeros((16,), jnp.float32)
    for j in range(_GEOM // 16):
        part[pl.ds(16 * j, 16)] = zero
    for h in hs:
        h.wait()

    for g in range(_WROWS // 16):
        pf_v = [pcb[pl.ds((12 + d) * _WROWS + 16 * g, 16)] for d in range(3)]
        pe = [pcb[pl.ds((s * 3 + d) * _WROWS + 16 * g, 16)] - pf_v[d]
              for s in range(4) for d in range(3)]
        ssq = [pe[3 * s] * pe[3 * s] + pe[3 * s + 1] * pe[3 * s + 1]
               + pe[3 * s + 2] * pe[3 * s + 2] for s in range(4)]
        na = [_sc_sqrt(q) for q in ssq]
        for s in range(4):
            sp = (s - 1) % 4
            dot = (pe[3 * sp] * pe[3 * s] + pe[3 * sp + 1] * pe[3 * s + 1]
                   + pe[3 * sp + 2] * pe[3 * s + 2])
            cosv = dot / jnp.maximum(na[sp] * na[s], jnp.float32(1e-8))
            plsc.addupdate(part.at[pl.ds(16 * s, 16)], cosv)
            plsc.addupdate(part.at[pl.ds(64 + 16 * s, 16)], na[s])
            plsc.addupdate(part.at[pl.ds(128 + 16 * s, 16)], cosv * cosv)
            plsc.addupdate(part.at[pl.ds(192 + 16 * s, 16)], na[s] * na[s])
    pltpu.sync_copy(part, out_hbm.at[pl.ds(wid * _GEOM, _GEOM)])


def _sc_geom(pc1):
    mesh = plsc.VectorSubcoreMesh(core_axis_name="c", subcore_axis_name="s")
    f = pl.kernel(
        _sc_geom_body,
        mesh=mesh,
        out_type=jax.ShapeDtypeStruct((32 * _GEOM,), jnp.float32),
        scratch_types=[
            pltpu.VMEM((15 * _WROWS,), jnp.float32),
            pltpu.VMEM((_GEOM,), jnp.float32),
            pltpu.SemaphoreType.DMA,
        ],
    )
    return f(pc1)


def _tc_stats_body(xn_ref, xf_ref, ed_ref, out_ref, a_xn, a_xf, a_ed, a_sq):
    i = pl.program_id(0)

    @pl.when(i == 0)
    def _init():
        a_xn[...] = jnp.zeros_like(a_xn)
        a_xf[...] = jnp.zeros_like(a_xf)
        a_ed[...] = jnp.zeros_like(a_ed)
        for k in range(3):
            a_sq[k] = 0.0

    xn = xn_ref[...]
    xf = xf_ref[...]
    ed = ed_ref[...]
    a_xn[...] += jnp.sum(xn, axis=0, keepdims=True)
    a_xf[...] += jnp.sum(xf, axis=0, keepdims=True)
    a_ed[...] += jnp.sum(ed, axis=0, keepdims=True)
    a_sq[0] = a_sq[0] + jnp.sum(xn * xn)
    a_sq[1] = a_sq[1] + jnp.sum(xf * xf)
    a_sq[2] = a_sq[2] + jnp.sum(ed * ed)

    @pl.when(i == _G - 1)
    def _fin():
        li = lax.broadcasted_iota(jnp.int32, (1, 64), 1)
        srow = (jnp.where(li == 0, a_sq[0], 0.0)
                + jnp.where(li == 1, a_sq[1], 0.0)
                + jnp.where(li == 2, a_sq[2], 0.0))
        out_ref[...] = jnp.concatenate(
            [a_xn[...], a_xf[...], a_ed[...], srow], axis=-1)


def _tc_stats(xn2, xf, ed2):
    return pl.pallas_call(
        _tc_stats_body,
        grid=(_G,),
        in_specs=[
            pl.BlockSpec((_BLK, _S * _ND), lambda i: (i, 0)),
            pl.BlockSpec((_BLK, _ND), lambda i: (i, 0)),
            pl.BlockSpec((_BLK, _S * _ED), lambda i: (i, 0)),
        ],
        out_specs=pl.BlockSpec((1, 768), lambda i: (0, 0)),
        out_shape=jax.ShapeDtypeStruct((1, 768), jnp.float32),
        scratch_shapes=[
            pltpu.VMEM((1, _S * _ND), jnp.float32),
            pltpu.VMEM((1, _ND), jnp.float32),
            pltpu.VMEM((1, _S * _ED), jnp.float32),
            pltpu.SMEM((4,), jnp.float32),
        ],
    )(xn2, xf, ed2)


def _intra_cols(p12):
    cur = [p12[:, 3 * s:3 * s + 3] for s in range(_S)]
    ssq = [jnp.sum(c * c, axis=-1, keepdims=True) for c in cur]
    na = [jnp.sqrt(q) for q in ssq]
    intra = []
    for s in range(_S):
        sp = (s - 1) % _S
        dot = jnp.sum(cur[sp] * cur[s], axis=-1, keepdims=True)
        intra.append(dot / jnp.maximum(na[sp] * na[s], 1e-8))
    return jnp.concatenate(intra, axis=-1), jnp.concatenate(na, axis=-1)


def _epi_body(st1_ref, st2_ref, pxs_ref, ped_ref, pps_ref, xc_ref, out_ref):
    nf = jnp.float32(_N)
    s1 = st1_ref[...]
    s_xn = s1[:, 0:512]
    s_xf = s1[:, 512:640]
    s_ed = s1[:, 640:704]
    q_xn = jnp.sum(s1[:, 704:705])
    q_xf = jnp.sum(s1[:, 705:706])
    q_ed = jnp.sum(s1[:, 706:707])
    s2 = jnp.sum(st2_ref[...], axis=0, keepdims=True)
    a_in = [jnp.sum(s2[:, 16 * k:16 * k + 16]) for k in range(4)]
    a_ln = [jnp.sum(s2[:, 64 + 16 * k:80 + 16 * k]) for k in range(4)]
    q_in = jnp.sum(s2[:, 128:192])
    q_ln = jnp.sum(s2[:, 192:256])

    iota = lax.broadcasted_iota(jnp.int32, (_P, 1), 0)
    ot = jnp.zeros((1, _L), jnp.float32)
    oi = lax.broadcasted_iota(jnp.int32, (1, _L), 1)
    for l in range(_L):
        pxs = pxs_ref[pl.ds(_P * l, _P), :]
        ped = ped_ref[pl.ds(_P * l, _P), :]
        pps = pps_ref[pl.ds(_P * l, _P), :]

        b_in, b_ln = _intra_cols(pps)
        cr_in = sum(b_in[:, k:k + 1] * a_in[k] for k in range(4))
        cr_ln = sum(b_ln[:, k:k + 1] * a_ln[k] for k in range(4))
        t_ang = (q_in - 2.0 * cr_in
                 + nf * jnp.sum(b_in * b_in, -1, keepdims=True))
        t_len = (q_ln - 2.0 * cr_ln
                 + nf * jnp.sum(b_ln * b_ln, -1, keepdims=True))
        t_sup = (q_xn - 2.0 * jnp.sum(pxs * s_xn, -1, keepdims=True)
                 + nf * jnp.sum(pxs * pxs, -1, keepdims=True))
        t_edg = (q_ed - 2.0 * jnp.sum(ped * s_ed, -1, keepdims=True)
                 + nf * jnp.sum(ped * ped, -1, keepdims=True))

        tmin = jnp.min(t_ang)
        bidx = jnp.min(jnp.where(t_ang <= tmin, iota, _P))
        onehot = iota == bidx
        t_len_b = jnp.sum(jnp.where(onehot, t_len, 0.0))
        t_sup_b = jnp.sum(jnp.where(onehot, t_sup, 0.0))
        t_edg_b = jnp.sum(jnp.where(onehot, t_edg, 0.0))

        xc = xc_ref[pl.ds(l, 1), :]
        t_cen = q_xf - 2.0 * jnp.sum(xc * s_xf) + nf * jnp.sum(xc * xc)

        sc_ang = _atan_pos(1.0 / tmin)
        sc_len = _atan_pos(1.0 / t_len_b)
        sc_sup = _atan_pos(1.0 / t_sup_b)
        sc_cen = _atan_pos(1.0 / t_cen)
        sc_edg = _atan_pos(1.0 / t_edg_b)

        m = jnp.float32(_M)
        tot = ((sc_len - m) ** 2 + (sc_ang - m) ** 2 + (sc_sup - m) ** 2
               + (sc_cen - m) ** 2 + (sc_edg - m) ** 2)
        sc = _atan_pos(1.0 / tot)
        ot = ot + jnp.where(oi == l, sc, 0.0)
    out_ref[...] = ot


def _epilogue(stats1, stats2, pxs, ped, pps, xc2):
    return pl.pallas_call(
        _epi_body,
        out_shape=jax.ShapeDtypeStruct((1, _L), jnp.float32),
    )(stats1, stats2, pxs, ped, pps, xc2)


def kernel(x_focal, p_focal, x_neighbor, p_neighbor, edge_attr_neighbor,
           x_center, x_support, edge_attr_support, p_support):
    n = x_focal.shape[0]
    pc = jnp.concatenate([p_neighbor.reshape(n, _S * _D).T, p_focal.T], 0)
    pc1 = jnp.pad(pc, ((0, 0), (0, _NPAD - n))).reshape(-1)
    stats2 = _sc_geom(pc1).reshape(32, _GEOM)

    xn2 = x_neighbor.reshape(n, _S * _ND)
    ed2 = edge_attr_neighbor.reshape(n, _S * _ED)
    stats1 = _tc_stats(xn2, x_focal, ed2)

    pxs = x_support[:, _PERMS].reshape(_L * _P, _S * _ND)
    ped = edge_attr_support[:, _PERMS].reshape(_L * _P, _S * _ED)
    pps = p_support[:, _PERMS].reshape(_L * _P, _S * _D)
    xc2 = x_center.reshape(_L, _ND)
    out = _epilogue(stats1, stats2, pxs, ped, pps, xc2)
    return out.reshape(_L)
